# baseline (speedup 1.0000x reference)
import functools

import jax
import jax.numpy as jnp
from jax import lax
from jax.experimental import pallas as pl
from jax.experimental.pallas import tpu as pltpu
from jax.experimental.pallas import tpu_sc as plsc

DIM = 64
PDIM = 128
NC = 2
NS = 16
NW = NC * NS

NBG = 8
NCHUNK = 4

TBLK = 32768


def _emb_lookup_chunk(idx, table, chunk_b, chunk_nb):
    mesh = plsc.VectorSubcoreMesh(core_axis_name="c", subcore_axis_name="s")
    _, seq = idx.shape
    b_per_w = chunk_nb // NW
    n_groups = b_per_w // NBG

    @functools.partial(
        pl.kernel,
        mesh=mesh,
        out_type=jax.ShapeDtypeStruct((chunk_nb, seq, PDIM), jnp.float32),
        scratch_types=[
            pltpu.VMEM((2, NBG, seq), jnp.int32),
            pltpu.VMEM((2, NBG, seq, PDIM), jnp.float32),
            pltpu.SemaphoreType.DMA,
            pltpu.SemaphoreType.DMA,
        ],
    )
    def body(idx_hbm, table_hbm, out_hbm, idx_v, rows_v, sem0, sem1):
        wid = lax.axis_index("s") * NC + lax.axis_index("c")
        base_b = wid * b_per_w
        sems = (sem0, sem1)

        def stage_and_fire(g, buf):
            pltpu.sync_copy(
                idx_hbm.at[pl.ds(chunk_b + base_b + g * NBG, NBG)],
                idx_v.at[buf],
            )
            sem = sems[buf]
            for i in range(NBG):
                pltpu.async_copy(
                    table_hbm.at[idx_v.at[buf, i]],
                    rows_v.at[buf, i],
                    sem,
                )

        def drain_and_store(g, buf):
            sem = sems[buf]
            for i in range(NBG):
                pltpu.make_async_copy(
                    table_hbm.at[idx_v.at[buf, i]], rows_v.at[buf, i], sem
                ).wait()
            pltpu.sync_copy(
                rows_v.at[buf], out_hbm.at[pl.ds(base_b + g * NBG, NBG)]
            )

        stage_and_fire(0, 0)
        n_outer = n_groups // 2

        def outer(t, carry):
            g0 = 2 * t
            stage_and_fire(g0 + 1, 1)
            drain_and_store(g0, 0)

            @pl.when(t + 1 < n_outer)
            def _():
                stage_and_fire(g0 + 2, 0)

            drain_and_store(g0 + 1, 1)
            return carry

        lax.fori_loop(0, n_outer, outer, 0)

    return body(idx, table)


def _tpad_body(emb_t_ref, out_ref):
    out_ref[:, :DIM] = emb_t_ref[...].T


def _transpose_pad(emb_t):
    nrows = emb_t.shape[1]
    grid = (nrows + TBLK - 1) // TBLK
    return pl.pallas_call(
        _tpad_body,
        grid=(grid,),
        in_specs=[pl.BlockSpec((DIM, TBLK), lambda i: (0, i))],
        out_specs=pl.BlockSpec((TBLK, PDIM), lambda i: (i, 0)),
        out_shape=jax.ShapeDtypeStruct((nrows, PDIM), jnp.float32),
    )(emb_t)


OB = 512


def _otrans_first_body(g_ref, out_ref):
    out_ref[...] = jnp.transpose(g_ref[:, :, :DIM], (1, 2, 0))


def _otrans_alias_body(g_ref, prev_ref, out_ref):
    del prev_ref
    out_ref[...] = jnp.transpose(g_ref[:, :, :DIM], (1, 2, 0))


def _out_transpose(chunks, nbatch, seq):
    cb = chunks[0].shape[0]
    nob = cb // OB
    out_shape = jax.ShapeDtypeStruct((seq, DIM, nbatch), jnp.float32)
    g_spec = pl.BlockSpec((OB, seq, PDIM), lambda i: (i, 0, 0))

    out = pl.pallas_call(
        _otrans_first_body,
        grid=(nob,),
        in_specs=[g_spec],
        out_specs=pl.BlockSpec((seq, DIM, OB), lambda i: (0, 0, i)),
        out_shape=out_shape,
    )(chunks[0])

    for c in range(1, len(chunks)):
        out = pl.pallas_call(
            _otrans_alias_body,
            grid=(nob,),
            in_specs=[
                g_spec,
                pl.BlockSpec(memory_space=pltpu.MemorySpace.HBM),
            ],
            out_specs=pl.BlockSpec(
                (seq, DIM, OB), lambda i, c=c: (0, 0, c * nob + i)
            ),
            out_shape=out_shape,
            input_output_aliases={1: 0},
        )(chunks[c], out)
    return out


def kernel(token_ids, embeddings):
    nbatch, seq = token_ids.shape
    table_p = _transpose_pad(embeddings.T)
    tids = token_ids.astype(jnp.int32)

    cb = nbatch // NCHUNK
    chunks = [
        _emb_lookup_chunk(tids, table_p, c * cb, cb) for c in range(NCHUNK)
    ]
    out_t = _out_transpose(chunks, nbatch, seq)
    return jnp.transpose(out_t, (2, 0, 1))


# speedup vs baseline: 1.0736x; 1.0736x over previous
import functools

import jax
import jax.numpy as jnp
from jax import lax
from jax.experimental import pallas as pl
from jax.experimental.pallas import tpu as pltpu
from jax.experimental.pallas import tpu_sc as plsc

DIM = 64
PDIM = 128
NC = 2
NS = 16
NW = NC * NS

NBG = 8


def _emb_lookup(idx, table):
    mesh = plsc.VectorSubcoreMesh(core_axis_name="c", subcore_axis_name="s")
    nbatch, seq = idx.shape
    b_per_w = nbatch // NW
    n_groups = b_per_w // NBG

    @functools.partial(
        pl.kernel,
        mesh=mesh,
        out_type=jax.ShapeDtypeStruct((nbatch, seq, PDIM), jnp.float32),
        scratch_types=[
            pltpu.VMEM((2, NBG, seq), jnp.int32),
            pltpu.VMEM((2, NBG, seq, PDIM), jnp.float32),
            pltpu.SemaphoreType.DMA,
            pltpu.SemaphoreType.DMA,
        ],
    )
    def body(idx_hbm, table_hbm, out_hbm, idx_v, rows_v, sem0, sem1):
        wid = lax.axis_index("s") * NC + lax.axis_index("c")
        base_b = wid * b_per_w
        sems = (sem0, sem1)

        def stage_and_fire(g, buf):
            pltpu.sync_copy(
                idx_hbm.at[pl.ds(base_b + g * NBG, NBG)], idx_v.at[buf]
            )
            sem = sems[buf]
            for i in range(NBG):
                pltpu.async_copy(
                    table_hbm.at[idx_v.at[buf, i]],
                    rows_v.at[buf, i],
                    sem,
                )

        def drain_and_store(g, buf):
            sem = sems[buf]
            for i in range(NBG):
                pltpu.make_async_copy(
                    table_hbm.at[idx_v.at[buf, i]], rows_v.at[buf, i], sem
                ).wait()
            pltpu.sync_copy(
                rows_v.at[buf], out_hbm.at[pl.ds(base_b + g * NBG, NBG)]
            )

        stage_and_fire(0, 0)
        n_outer = n_groups // 2

        def outer(t, carry):
            g0 = 2 * t
            stage_and_fire(g0 + 1, 1)
            drain_and_store(g0, 0)

            @pl.when(t + 1 < n_outer)
            def _():
                stage_and_fire(g0 + 2, 0)

            drain_and_store(g0 + 1, 1)
            return carry

        lax.fori_loop(0, n_outer, outer, 0)

    return body(idx, table)


TBLK = 32768


def _tpad_body(emb_t_ref, out_ref):
    out_ref[:, :DIM] = emb_t_ref[...].T


def _transpose_pad(emb_t):
    nrows = emb_t.shape[1]
    grid = (nrows + TBLK - 1) // TBLK
    return pl.pallas_call(
        _tpad_body,
        grid=(grid,),
        in_specs=[pl.BlockSpec((DIM, TBLK), lambda i: (0, i))],
        out_specs=pl.BlockSpec((TBLK, PDIM), lambda i: (i, 0)),
        out_shape=jax.ShapeDtypeStruct((nrows, PDIM), jnp.float32),
    )(emb_t)


def kernel(token_ids, embeddings):
    table_p = _transpose_pad(embeddings.T)
    out_p = _emb_lookup(token_ids.astype(jnp.int32), table_p)
    return out_p[:, :, :DIM]
